# Initial kernel scaffold; baseline (speedup 1.0000x reference)
import jax
import jax.numpy as jnp
from jax.experimental import pallas as pl


def kernel(seq_in, embeddings, W_ih0, W_hh0, b_ih0, b_hh0, W_ih1, W_hh1, b_ih1, b_hh1, W_fc, b_fc):
    raise NotImplementedError("write your pallas kernel here")


# speedup vs baseline: 1.0411x; 1.0411x over previous
import functools

import jax
import jax.numpy as jnp
from jax import lax
from jax.experimental import pallas as pl
from jax.experimental.pallas import tpu as pltpu
from jax.experimental.pallas import tpu_sc as plsc

N_VOCAB = 100000
HID = 128
EMB = 64
B = 1024
L = 50
NTOK = B * L


def _sc_gather(idx_flat, table):
    info = plsc.get_sparse_core_info()
    NC, NS = info.num_cores, info.num_subcores
    NW = NC * NS
    n_per_w = NTOK // NW
    CH = 80
    NCH = n_per_w // CH
    assert NCH * CH == n_per_w and NTOK % NW == 0

    mesh = plsc.VectorSubcoreMesh(core_axis_name="c", subcore_axis_name="s")

    @functools.partial(
        pl.kernel,
        out_type=jax.ShapeDtypeStruct((NTOK, EMB), jnp.float32),
        mesh=mesh,
        scratch_types=[
            pltpu.VMEM((NCH, CH), jnp.int32),
            pltpu.VMEM((n_per_w, EMB), jnp.float32),
            pltpu.SemaphoreType.DMA,
        ],
        compiler_params=pltpu.CompilerParams(use_tc_tiling_on_sc=False),
    )
    def gather_k(idx_hbm, table_hbm, out_hbm, idx_v, rows_v, sem):
        wid = lax.axis_index("s") * NC + lax.axis_index("c")
        base = wid * n_per_w
        pltpu.sync_copy(idx_hbm.at[wid], idx_v)
        copies = []
        for j in range(NCH):
            copies.append(
                pltpu.async_copy(
                    table_hbm.at[idx_v.at[j]],
                    rows_v.at[pl.ds(j * CH, CH)],
                    sem,
                )
            )
        for c in copies:
            c.wait()
        pltpu.sync_copy(rows_v, out_hbm.at[pl.ds(base, n_per_w)])

    return gather_k(idx_flat.reshape(NW, NCH, CH), table)


def _lstm_body(x_ref, wih0_ref, whh0_ref, b0_ref, wih1_ref, whh1_ref, b1_ref,
               out_ref, h0, c0, h1, c1):
    t = pl.program_id(0)

    @pl.when(t == 0)
    def _():
        h0[...] = jnp.zeros_like(h0)
        c0[...] = jnp.zeros_like(c0)
        h1[...] = jnp.zeros_like(h1)
        c1[...] = jnp.zeros_like(c1)

    x = x_ref[0]
    g0 = (
        jnp.dot(x, wih0_ref[...], preferred_element_type=jnp.float32)
        + jnp.dot(h0[...], whh0_ref[...], preferred_element_type=jnp.float32)
        + b0_ref[...]
    )
    i0 = jax.nn.sigmoid(g0[:, 0:HID])
    f0 = jax.nn.sigmoid(g0[:, HID:2 * HID])
    gg0 = jnp.tanh(g0[:, 2 * HID:3 * HID])
    o0 = jax.nn.sigmoid(g0[:, 3 * HID:4 * HID])
    cn0 = f0 * c0[...] + i0 * gg0
    hn0 = o0 * jnp.tanh(cn0)
    c0[...] = cn0
    h0[...] = hn0

    g1 = (
        jnp.dot(hn0, wih1_ref[...], preferred_element_type=jnp.float32)
        + jnp.dot(h1[...], whh1_ref[...], preferred_element_type=jnp.float32)
        + b1_ref[...]
    )
    i1 = jax.nn.sigmoid(g1[:, 0:HID])
    f1 = jax.nn.sigmoid(g1[:, HID:2 * HID])
    gg1 = jnp.tanh(g1[:, 2 * HID:3 * HID])
    o1 = jax.nn.sigmoid(g1[:, 3 * HID:4 * HID])
    cn1 = f1 * c1[...] + i1 * gg1
    hn1 = o1 * jnp.tanh(cn1)
    c1[...] = cn1
    h1[...] = hn1

    @pl.when(t == L - 1)
    def _():
        out_ref[...] = hn1


def _lstm(embedded, wih0T, whh0T, b0, wih1T, whh1T, b1, *, interpret=False):
    return pl.pallas_call(
        _lstm_body,
        grid=(L,),
        in_specs=[
            pl.BlockSpec((1, B, EMB), lambda t: (t, 0, 0)),
            pl.BlockSpec((EMB, 4 * HID), lambda t: (0, 0)),
            pl.BlockSpec((HID, 4 * HID), lambda t: (0, 0)),
            pl.BlockSpec((1, 4 * HID), lambda t: (0, 0)),
            pl.BlockSpec((HID, 4 * HID), lambda t: (0, 0)),
            pl.BlockSpec((HID, 4 * HID), lambda t: (0, 0)),
            pl.BlockSpec((1, 4 * HID), lambda t: (0, 0)),
        ],
        out_specs=pl.BlockSpec((B, HID), lambda t: (0, 0)),
        out_shape=jax.ShapeDtypeStruct((B, HID), jnp.float32),
        scratch_shapes=[pltpu.VMEM((B, HID), jnp.float32)] * 4,
        interpret=interpret,
    )(embedded, wih0T, whh0T, b0, wih1T, whh1T, b1)


_TV = 2048


def _fc_body(ht_ref, w_ref, b_ref, out_ref):
    out_ref[...] = (
        lax.dot_general(
            ht_ref[...], w_ref[...],
            (((1,), (1,)), ((), ())),
            preferred_element_type=jnp.float32,
        )
        + b_ref[...]
    )


def _fc(ht, W_fc, b_fc2d, *, interpret=False):
    nv = pl.cdiv(N_VOCAB, _TV)
    return pl.pallas_call(
        _fc_body,
        grid=(nv,),
        in_specs=[
            pl.BlockSpec((B, HID), lambda v: (0, 0)),
            pl.BlockSpec((_TV, HID), lambda v: (v, 0)),
            pl.BlockSpec((1, _TV), lambda v: (0, v)),
        ],
        out_specs=pl.BlockSpec((B, _TV), lambda v: (0, v)),
        out_shape=jax.ShapeDtypeStruct((B, N_VOCAB), jnp.float32),
        interpret=interpret,
    )(ht, W_fc, b_fc2d)


def kernel(seq_in, embeddings, W_ih0, W_hh0, b_ih0, b_hh0,
           W_ih1, W_hh1, b_ih1, b_hh1, W_fc, b_fc):
    idx_flat = seq_in.T.reshape(-1).astype(jnp.int32)
    emb_flat = _sc_gather(idx_flat, embeddings)
    embedded = emb_flat.reshape(L, B, EMB)

    b0 = (b_ih0 + b_hh0).reshape(1, 4 * HID)
    b1 = (b_ih1 + b_hh1).reshape(1, 4 * HID)
    ht = _lstm(embedded, W_ih0.T, W_hh0.T, b0, W_ih1.T, W_hh1.T, b1)

    return _fc(ht, W_fc, b_fc.reshape(1, N_VOCAB))


# speedup vs baseline: 1.0679x; 1.0257x over previous
import functools

import jax
import jax.numpy as jnp
from jax import lax
from jax.experimental import pallas as pl
from jax.experimental.pallas import tpu as pltpu
from jax.experimental.pallas import tpu_sc as plsc

N_VOCAB = 100000
HID = 128
EMB = 64
B = 1024
L = 50
NTOK = B * L


_D = 128


def _sc_gather(idx_flat, table_p):
    info = plsc.get_sparse_core_info()
    NC, NS = info.num_cores, info.num_subcores
    NW = NC * NS
    n_per_w = NTOK // NW
    CH = 80
    NCH = n_per_w // CH
    HALF = NCH // 2
    assert NCH * CH == n_per_w and NTOK % NW == 0 and NCH % 2 == 0

    mesh = plsc.VectorSubcoreMesh(core_axis_name="c", subcore_axis_name="s")

    @functools.partial(
        pl.kernel,
        out_type=jax.ShapeDtypeStruct((NTOK, _D), jnp.float32),
        mesh=mesh,
        scratch_types=[
            pltpu.VMEM((NCH, CH), jnp.int32),
            pltpu.VMEM((HALF * CH, _D), jnp.float32),
            pltpu.SemaphoreType.DMA,
        ],
        compiler_params=pltpu.CompilerParams(use_tc_tiling_on_sc=False),
    )
    def gather_k(idx_hbm, table_hbm, out_hbm, idx_v, rows_v, sem):
        wid = lax.axis_index("s") * NC + lax.axis_index("c")
        base = wid * n_per_w
        pltpu.sync_copy(idx_hbm.at[wid], idx_v)
        for h in range(2):
            copies = []
            for j in range(HALF):
                copies.append(
                    pltpu.async_copy(
                        table_hbm.at[idx_v.at[h * HALF + j]],
                        rows_v.at[pl.ds(j * CH, CH)],
                        sem,
                    )
                )
            for c in copies:
                c.wait()
            pltpu.sync_copy(
                rows_v, out_hbm.at[pl.ds(base + h * HALF * CH, HALF * CH)]
            )

    return gather_k(idx_flat.reshape(NW, NCH, CH), table_p)


def _lstm_body(x_ref, wih0_ref, whh0_ref, b0_ref, wih1_ref, whh1_ref, b1_ref,
               out_ref, h0, c0, h1, c1):
    t = pl.program_id(0)

    @pl.when(t == 0)
    def _():
        h0[...] = jnp.zeros_like(h0)
        c0[...] = jnp.zeros_like(c0)
        h1[...] = jnp.zeros_like(h1)
        c1[...] = jnp.zeros_like(c1)

    x = x_ref[0]
    g0 = (
        jnp.dot(x, wih0_ref[...], preferred_element_type=jnp.float32)
        + jnp.dot(h0[...], whh0_ref[...], preferred_element_type=jnp.float32)
        + b0_ref[...]
    )
    i0 = jax.nn.sigmoid(g0[:, 0:HID])
    f0 = jax.nn.sigmoid(g0[:, HID:2 * HID])
    gg0 = jnp.tanh(g0[:, 2 * HID:3 * HID])
    o0 = jax.nn.sigmoid(g0[:, 3 * HID:4 * HID])
    cn0 = f0 * c0[...] + i0 * gg0
    hn0 = o0 * jnp.tanh(cn0)
    c0[...] = cn0
    h0[...] = hn0

    g1 = (
        jnp.dot(hn0, wih1_ref[...], preferred_element_type=jnp.float32)
        + jnp.dot(h1[...], whh1_ref[...], preferred_element_type=jnp.float32)
        + b1_ref[...]
    )
    i1 = jax.nn.sigmoid(g1[:, 0:HID])
    f1 = jax.nn.sigmoid(g1[:, HID:2 * HID])
    gg1 = jnp.tanh(g1[:, 2 * HID:3 * HID])
    o1 = jax.nn.sigmoid(g1[:, 3 * HID:4 * HID])
    cn1 = f1 * c1[...] + i1 * gg1
    hn1 = o1 * jnp.tanh(cn1)
    c1[...] = cn1
    h1[...] = hn1

    @pl.when(t == L - 1)
    def _():
        out_ref[...] = hn1


def _lstm(embedded, wih0T, whh0T, b0, wih1T, whh1T, b1, *, interpret=False):
    return pl.pallas_call(
        _lstm_body,
        grid=(L,),
        in_specs=[
            pl.BlockSpec((1, B, _D), lambda t: (t, 0, 0)),
            pl.BlockSpec((_D, 4 * HID), lambda t: (0, 0)),
            pl.BlockSpec((HID, 4 * HID), lambda t: (0, 0)),
            pl.BlockSpec((1, 4 * HID), lambda t: (0, 0)),
            pl.BlockSpec((HID, 4 * HID), lambda t: (0, 0)),
            pl.BlockSpec((HID, 4 * HID), lambda t: (0, 0)),
            pl.BlockSpec((1, 4 * HID), lambda t: (0, 0)),
        ],
        out_specs=pl.BlockSpec((B, HID), lambda t: (0, 0)),
        out_shape=jax.ShapeDtypeStruct((B, HID), jnp.float32),
        scratch_shapes=[pltpu.VMEM((B, HID), jnp.float32)] * 4,
        interpret=interpret,
    )(embedded, wih0T, whh0T, b0, wih1T, whh1T, b1)


_TV = 2048


def _fc_body(ht_ref, w_ref, b_ref, out_ref):
    out_ref[...] = (
        lax.dot_general(
            ht_ref[...], w_ref[...],
            (((1,), (1,)), ((), ())),
            preferred_element_type=jnp.float32,
        )
        + b_ref[...]
    )


def _fc(ht, W_fc, b_fc2d, *, interpret=False):
    nv = pl.cdiv(N_VOCAB, _TV)
    return pl.pallas_call(
        _fc_body,
        grid=(nv,),
        in_specs=[
            pl.BlockSpec((B, HID), lambda v: (0, 0)),
            pl.BlockSpec((_TV, HID), lambda v: (v, 0)),
            pl.BlockSpec((1, _TV), lambda v: (0, v)),
        ],
        out_specs=pl.BlockSpec((B, _TV), lambda v: (0, v)),
        out_shape=jax.ShapeDtypeStruct((B, N_VOCAB), jnp.float32),
        interpret=interpret,
    )(ht, W_fc, b_fc2d)


def kernel(seq_in, embeddings, W_ih0, W_hh0, b_ih0, b_hh0,
           W_ih1, W_hh1, b_ih1, b_hh1, W_fc, b_fc):
    idx_flat = seq_in.T.reshape(-1).astype(jnp.int32)
    table_p = jnp.pad(embeddings, ((0, 0), (0, _D - EMB)))
    emb_flat = _sc_gather(idx_flat, table_p)
    embedded = emb_flat.reshape(L, B, _D)

    b0 = (b_ih0 + b_hh0).reshape(1, 4 * HID)
    b1 = (b_ih1 + b_hh1).reshape(1, 4 * HID)
    wih0T_p = jnp.pad(W_ih0.T, ((0, _D - EMB), (0, 0)))
    ht = _lstm(embedded, wih0T_p, W_hh0.T, b0, W_ih1.T, W_hh1.T, b1)

    return _fc(ht, W_fc, b_fc.reshape(1, N_VOCAB))


# speedup vs baseline: 1.0725x; 1.0043x over previous
import functools

import jax
import jax.numpy as jnp
from jax import lax
from jax.experimental import pallas as pl
from jax.experimental.pallas import tpu as pltpu
from jax.experimental.pallas import tpu_sc as plsc

N_VOCAB = 100000
HID = 128
EMB = 64
B = 1024
L = 50
NTOK = B * L


_D = 128


def _sc_gather(idx_flat, table_p):
    info = plsc.get_sparse_core_info()
    NC, NS = info.num_cores, info.num_subcores
    NW = NC * NS
    n_per_w = NTOK // NW
    CH = 80
    NCH = n_per_w // CH
    HALF = NCH // 2
    assert NCH * CH == n_per_w and NTOK % NW == 0 and NCH % 2 == 0

    mesh = plsc.VectorSubcoreMesh(core_axis_name="c", subcore_axis_name="s")

    @functools.partial(
        pl.kernel,
        out_type=jax.ShapeDtypeStruct((NTOK, _D), jnp.float32),
        mesh=mesh,
        scratch_types=[
            pltpu.VMEM((NCH, CH), jnp.int32),
            pltpu.VMEM((HALF * CH, _D), jnp.float32),
            pltpu.SemaphoreType.DMA,
        ],
        compiler_params=pltpu.CompilerParams(use_tc_tiling_on_sc=True),
    )
    def gather_k(idx_hbm, table_hbm, out_hbm, idx_v, rows_v, sem):
        wid = lax.axis_index("s") * NC + lax.axis_index("c")
        base = wid * n_per_w
        pltpu.sync_copy(idx_hbm.at[wid], idx_v)
        for h in range(2):
            copies = []
            for j in range(HALF):
                copies.append(
                    pltpu.async_copy(
                        table_hbm.at[idx_v.at[h * HALF + j]],
                        rows_v.at[pl.ds(j * CH, CH)],
                        sem,
                    )
                )
            for c in copies:
                c.wait()
            pltpu.sync_copy(
                rows_v, out_hbm.at[pl.ds(base + h * HALF * CH, HALF * CH)]
            )

    return gather_k(idx_flat.reshape(NW, NCH, CH), table_p)


def _lstm_body(x_ref, wih0_ref, whh0_ref, b0_ref, wih1_ref, whh1_ref, b1_ref,
               out_ref, h0, c0, h1, c1):
    t = pl.program_id(0)

    @pl.when(t == 0)
    def _():
        h0[...] = jnp.zeros_like(h0)
        c0[...] = jnp.zeros_like(c0)
        h1[...] = jnp.zeros_like(h1)
        c1[...] = jnp.zeros_like(c1)

    x = x_ref[0]
    g0 = (
        jnp.dot(x, wih0_ref[...], preferred_element_type=jnp.float32)
        + jnp.dot(h0[...], whh0_ref[...], preferred_element_type=jnp.float32)
        + b0_ref[...]
    )
    i0 = jax.nn.sigmoid(g0[:, 0:HID])
    f0 = jax.nn.sigmoid(g0[:, HID:2 * HID])
    gg0 = jnp.tanh(g0[:, 2 * HID:3 * HID])
    o0 = jax.nn.sigmoid(g0[:, 3 * HID:4 * HID])
    cn0 = f0 * c0[...] + i0 * gg0
    hn0 = o0 * jnp.tanh(cn0)
    c0[...] = cn0
    h0[...] = hn0

    g1 = (
        jnp.dot(hn0, wih1_ref[...], preferred_element_type=jnp.float32)
        + jnp.dot(h1[...], whh1_ref[...], preferred_element_type=jnp.float32)
        + b1_ref[...]
    )
    i1 = jax.nn.sigmoid(g1[:, 0:HID])
    f1 = jax.nn.sigmoid(g1[:, HID:2 * HID])
    gg1 = jnp.tanh(g1[:, 2 * HID:3 * HID])
    o1 = jax.nn.sigmoid(g1[:, 3 * HID:4 * HID])
    cn1 = f1 * c1[...] + i1 * gg1
    hn1 = o1 * jnp.tanh(cn1)
    c1[...] = cn1
    h1[...] = hn1

    @pl.when(t == L - 1)
    def _():
        out_ref[...] = hn1


def _lstm(embedded, wih0T, whh0T, b0, wih1T, whh1T, b1, *, interpret=False):
    return pl.pallas_call(
        _lstm_body,
        grid=(L,),
        in_specs=[
            pl.BlockSpec((1, B, _D), lambda t: (t, 0, 0)),
            pl.BlockSpec((_D, 4 * HID), lambda t: (0, 0)),
            pl.BlockSpec((HID, 4 * HID), lambda t: (0, 0)),
            pl.BlockSpec((1, 4 * HID), lambda t: (0, 0)),
            pl.BlockSpec((HID, 4 * HID), lambda t: (0, 0)),
            pl.BlockSpec((HID, 4 * HID), lambda t: (0, 0)),
            pl.BlockSpec((1, 4 * HID), lambda t: (0, 0)),
        ],
        out_specs=pl.BlockSpec((B, HID), lambda t: (0, 0)),
        out_shape=jax.ShapeDtypeStruct((B, HID), jnp.float32),
        scratch_shapes=[pltpu.VMEM((B, HID), jnp.float32)] * 4,
        interpret=interpret,
    )(embedded, wih0T, whh0T, b0, wih1T, whh1T, b1)


_TV = 2048


def _fc_body(ht_ref, w_ref, b_ref, out_ref):
    out_ref[...] = (
        lax.dot_general(
            ht_ref[...], w_ref[...],
            (((1,), (1,)), ((), ())),
            preferred_element_type=jnp.float32,
        )
        + b_ref[...]
    )


def _fc(ht, W_fc, b_fc2d, *, interpret=False):
    nv = pl.cdiv(N_VOCAB, _TV)
    return pl.pallas_call(
        _fc_body,
        grid=(nv,),
        in_specs=[
            pl.BlockSpec((B, HID), lambda v: (0, 0)),
            pl.BlockSpec((_TV, HID), lambda v: (v, 0)),
            pl.BlockSpec((1, _TV), lambda v: (0, v)),
        ],
        out_specs=pl.BlockSpec((B, _TV), lambda v: (0, v)),
        out_shape=jax.ShapeDtypeStruct((B, N_VOCAB), jnp.float32),
        interpret=interpret,
    )(ht, W_fc, b_fc2d)


def kernel(seq_in, embeddings, W_ih0, W_hh0, b_ih0, b_hh0,
           W_ih1, W_hh1, b_ih1, b_hh1, W_fc, b_fc):
    idx_flat = seq_in.T.reshape(-1).astype(jnp.int32)
    table_p = jnp.pad(embeddings, ((0, 0), (0, _D - EMB)))
    emb_flat = _sc_gather(idx_flat, table_p)
    embedded = emb_flat.reshape(L, B, _D)

    b0 = (b_ih0 + b_hh0).reshape(1, 4 * HID)
    b1 = (b_ih1 + b_hh1).reshape(1, 4 * HID)
    wih0T_p = jnp.pad(W_ih0.T, ((0, _D - EMB), (0, 0)))
    ht = _lstm(embedded, wih0T_p, W_hh0.T, b0, W_ih1.T, W_hh1.T, b1)

    return _fc(ht, W_fc, b_fc.reshape(1, N_VOCAB))


# speedup vs baseline: 2.2187x; 2.0688x over previous
import functools

import jax
import jax.numpy as jnp
from jax import lax
from jax.experimental import pallas as pl
from jax.experimental.pallas import tpu as pltpu
from jax.experimental.pallas import tpu_sc as plsc

N_VOCAB = 100000
HID = 128
EMB = 64
B = 1024
L = 50
NTOK = B * L


_D = 128


def _sc_gather(idx_flat, table_p):
    info = plsc.get_sparse_core_info()
    NC, NS = info.num_cores, info.num_subcores
    NW = NC * NS
    n_per_w = NTOK // NW
    CH = 80
    NCH = n_per_w // CH
    HALF = NCH // 2
    assert NCH * CH == n_per_w and NTOK % NW == 0 and NCH % 2 == 0

    mesh = plsc.VectorSubcoreMesh(core_axis_name="c", subcore_axis_name="s")

    @functools.partial(
        pl.kernel,
        out_type=jax.ShapeDtypeStruct((NTOK, _D), jnp.float32),
        mesh=mesh,
        scratch_types=[
            pltpu.VMEM((NCH, CH), jnp.int32),
            pltpu.VMEM((HALF * CH, _D), jnp.float32),
            pltpu.SemaphoreType.DMA,
        ],
        compiler_params=pltpu.CompilerParams(use_tc_tiling_on_sc=True),
    )
    def gather_k(idx_hbm, table_hbm, out_hbm, idx_v, rows_v, sem):
        wid = lax.axis_index("s") * NC + lax.axis_index("c")
        base = wid * n_per_w
        pltpu.sync_copy(idx_hbm.at[wid], idx_v)
        for h in range(2):
            copies = []
            for j in range(HALF):
                copies.append(
                    pltpu.async_copy(
                        table_hbm.at[idx_v.at[h * HALF + j]],
                        rows_v.at[pl.ds(j * CH, CH)],
                        sem,
                    )
                )
            for c in copies:
                c.wait()
            pltpu.sync_copy(
                rows_v, out_hbm.at[pl.ds(base + h * HALF * CH, HALF * CH)]
            )

    return gather_k(idx_flat.reshape(NW, NCH, CH), table_p)


def _lstm_body(x_ref, wih0_ref, whh0_ref, b0_ref, wih1_ref, whh1_ref, b1_ref,
               out_ref, h0, c0, h1, c1):
    t = pl.program_id(0)

    @pl.when(t == 0)
    def _():
        h0[...] = jnp.zeros_like(h0)
        c0[...] = jnp.zeros_like(c0)
        h1[...] = jnp.zeros_like(h1)
        c1[...] = jnp.zeros_like(c1)

    x = x_ref[0]
    g0 = (
        jnp.dot(x, wih0_ref[...], preferred_element_type=jnp.float32)
        + jnp.dot(h0[...], whh0_ref[...], preferred_element_type=jnp.float32)
        + b0_ref[...]
    )
    i0 = jax.nn.sigmoid(g0[:, 0:HID])
    f0 = jax.nn.sigmoid(g0[:, HID:2 * HID])
    gg0 = jnp.tanh(g0[:, 2 * HID:3 * HID])
    o0 = jax.nn.sigmoid(g0[:, 3 * HID:4 * HID])
    cn0 = f0 * c0[...] + i0 * gg0
    hn0 = o0 * jnp.tanh(cn0)
    c0[...] = cn0
    h0[...] = hn0

    g1 = (
        jnp.dot(hn0, wih1_ref[...], preferred_element_type=jnp.float32)
        + jnp.dot(h1[...], whh1_ref[...], preferred_element_type=jnp.float32)
        + b1_ref[...]
    )
    i1 = jax.nn.sigmoid(g1[:, 0:HID])
    f1 = jax.nn.sigmoid(g1[:, HID:2 * HID])
    gg1 = jnp.tanh(g1[:, 2 * HID:3 * HID])
    o1 = jax.nn.sigmoid(g1[:, 3 * HID:4 * HID])
    cn1 = f1 * c1[...] + i1 * gg1
    hn1 = o1 * jnp.tanh(cn1)
    c1[...] = cn1
    h1[...] = hn1

    @pl.when(t == L - 1)
    def _():
        out_ref[...] = hn1


def _lstm(embedded, wih0T, whh0T, b0, wih1T, whh1T, b1, *, interpret=False):
    return pl.pallas_call(
        _lstm_body,
        grid=(L,),
        in_specs=[
            pl.BlockSpec((1, B, _D), lambda t: (t, 0, 0)),
            pl.BlockSpec((_D, 4 * HID), lambda t: (0, 0)),
            pl.BlockSpec((HID, 4 * HID), lambda t: (0, 0)),
            pl.BlockSpec((1, 4 * HID), lambda t: (0, 0)),
            pl.BlockSpec((HID, 4 * HID), lambda t: (0, 0)),
            pl.BlockSpec((HID, 4 * HID), lambda t: (0, 0)),
            pl.BlockSpec((1, 4 * HID), lambda t: (0, 0)),
        ],
        out_specs=pl.BlockSpec((B, HID), lambda t: (0, 0)),
        out_shape=jax.ShapeDtypeStruct((B, HID), jnp.float32),
        scratch_shapes=[pltpu.VMEM((B, HID), jnp.float32)] * 4,
        interpret=interpret,
    )(embedded, wih0T, whh0T, b0, wih1T, whh1T, b1)


_TV = 2048


def _fc_body(htT_ref, w_ref, b_ref, out_ref):
    out_ref[...] = (
        jnp.dot(w_ref[...], htT_ref[...], preferred_element_type=jnp.float32)
        + jnp.transpose(b_ref[...])
    )


def _fc(htT, W_fc, b_fc2d, *, interpret=False):
    nv = pl.cdiv(N_VOCAB, _TV)
    return pl.pallas_call(
        _fc_body,
        grid=(nv,),
        in_specs=[
            pl.BlockSpec((HID, B), lambda v: (0, 0)),
            pl.BlockSpec((_TV, HID), lambda v: (v, 0)),
            pl.BlockSpec((1, _TV), lambda v: (0, v)),
        ],
        out_specs=pl.BlockSpec((_TV, B), lambda v: (v, 0)),
        out_shape=jax.ShapeDtypeStruct((N_VOCAB, B), jnp.float32),
        interpret=interpret,
    )(htT, W_fc, b_fc2d)


def kernel(seq_in, embeddings, W_ih0, W_hh0, b_ih0, b_hh0,
           W_ih1, W_hh1, b_ih1, b_hh1, W_fc, b_fc):
    idx_flat = seq_in.T.reshape(-1).astype(jnp.int32)
    table_p = jnp.pad(embeddings, ((0, 0), (0, _D - EMB)))
    emb_flat = _sc_gather(idx_flat, table_p)
    embedded = emb_flat.reshape(L, B, _D)

    b0 = (b_ih0 + b_hh0).reshape(1, 4 * HID)
    b1 = (b_ih1 + b_hh1).reshape(1, 4 * HID)
    wih0T_p = jnp.pad(W_ih0.T, ((0, _D - EMB), (0, 0)))
    ht = _lstm(embedded, wih0T_p, W_hh0.T, b0, W_ih1.T, W_hh1.T, b1)

    outT = _fc(ht.T, W_fc, b_fc.reshape(1, N_VOCAB))
    return outT.T


# speedup vs baseline: 2.4665x; 1.1117x over previous
import functools

import jax
import jax.numpy as jnp
from jax import lax
from jax.experimental import pallas as pl
from jax.experimental.pallas import tpu as pltpu
from jax.experimental.pallas import tpu_sc as plsc

N_VOCAB = 100000
HID = 128
EMB = 64
B = 1024
L = 50
NTOK = B * L


_D = 128


def _sc_gather(idx_flat, table_p):
    n_tok = idx_flat.shape[0]
    info = plsc.get_sparse_core_info()
    NC, NS = info.num_cores, info.num_subcores
    NW = NC * NS
    n_per_w = n_tok // NW
    CH = 80
    NCH = n_per_w // CH
    HALF = NCH // 2
    assert NCH * CH == n_per_w and n_tok % NW == 0 and NCH % 2 == 0

    mesh = plsc.VectorSubcoreMesh(core_axis_name="c", subcore_axis_name="s")

    @functools.partial(
        pl.kernel,
        out_type=jax.ShapeDtypeStruct((n_tok, _D), jnp.float32),
        mesh=mesh,
        scratch_types=[
            pltpu.VMEM((NCH, CH), jnp.int32),
            pltpu.VMEM((HALF * CH, _D), jnp.float32),
            pltpu.SemaphoreType.DMA,
        ],
        compiler_params=pltpu.CompilerParams(use_tc_tiling_on_sc=True),
    )
    def gather_k(idx_hbm, table_hbm, out_hbm, idx_v, rows_v, sem):
        wid = lax.axis_index("s") * NC + lax.axis_index("c")
        base = wid * n_per_w
        pltpu.sync_copy(idx_hbm.at[wid], idx_v)
        for h in range(2):
            copies = []
            for j in range(HALF):
                copies.append(
                    pltpu.async_copy(
                        table_hbm.at[idx_v.at[h * HALF + j]],
                        rows_v.at[pl.ds(j * CH, CH)],
                        sem,
                    )
                )
            for c in copies:
                c.wait()
            pltpu.sync_copy(
                rows_v, out_hbm.at[pl.ds(base + h * HALF * CH, HALF * CH)]
            )

    return gather_k(idx_flat.reshape(NW, NCH, CH), table_p)


def _sig(x):
    return 0.5 * jnp.tanh(0.5 * x) + 0.5


_LH = 10


def _make_lstm_body(n_steps, first):
    def body(*refs):
        if first:
            (x_ref, wih0_ref, b0_ref, wih1_ref, b1_ref,
             oh0, oc0, oh1, oc1, h0, c0, h1, c1) = refs
        else:
            (x_ref, wih0_ref, b0_ref, wih1_ref, b1_ref,
             ih0, ic0, ih1, ic1,
             oh0, oc0, oh1, oc1, h0, c0, h1, c1) = refs
        t = pl.program_id(0)

        @pl.when(t == 0)
        def _():
            if first:
                h0[...] = jnp.zeros_like(h0)
                c0[...] = jnp.zeros_like(c0)
                h1[...] = jnp.zeros_like(h1)
                c1[...] = jnp.zeros_like(c1)
            else:
                h0[...] = ih0[...]
                c0[...] = ic0[...]
                h1[...] = ih1[...]
                c1[...] = ic1[...]

        x = x_ref[0][:, :EMB].astype(jnp.bfloat16)
        xh = jnp.concatenate([x, h0[...].astype(jnp.bfloat16)], axis=1)
        g0 = (
            jnp.dot(xh, wih0_ref[...], preferred_element_type=jnp.float32)
            + b0_ref[...]
        )
        i0 = _sig(g0[:, 0:HID])
        f0 = _sig(g0[:, HID:2 * HID])
        gg0 = jnp.tanh(g0[:, 2 * HID:3 * HID])
        o0 = _sig(g0[:, 3 * HID:4 * HID])
        cn0 = f0 * c0[...] + i0 * gg0
        hn0 = o0 * jnp.tanh(cn0)
        c0[...] = cn0
        h0[...] = hn0

        hh = jnp.concatenate(
            [hn0.astype(jnp.bfloat16), h1[...].astype(jnp.bfloat16)], axis=1)
        g1 = (
            jnp.dot(hh, wih1_ref[...], preferred_element_type=jnp.float32)
            + b1_ref[...]
        )
        i1 = _sig(g1[:, 0:HID])
        f1 = _sig(g1[:, HID:2 * HID])
        gg1 = jnp.tanh(g1[:, 2 * HID:3 * HID])
        o1 = _sig(g1[:, 3 * HID:4 * HID])
        cn1 = f1 * c1[...] + i1 * gg1
        hn1 = o1 * jnp.tanh(cn1)
        c1[...] = cn1
        h1[...] = hn1

        @pl.when(t == n_steps - 1)
        def _():
            oh0[...] = hn0
            oc0[...] = cn0
            oh1[...] = hn1
            oc1[...] = cn1

    return body


def _lstm_part(emb, weights, carry=None, *, interpret=False):
    n_steps = emb.shape[0]
    first = carry is None
    st = jax.ShapeDtypeStruct((B, HID), jnp.float32)
    st_spec = pl.BlockSpec((B, HID), lambda t: (0, 0))
    in_specs = [
        pl.BlockSpec((1, B, _D), lambda t: (t, 0, 0)),
        pl.BlockSpec((EMB + HID, 4 * HID), lambda t: (0, 0)),
        pl.BlockSpec((1, 4 * HID), lambda t: (0, 0)),
        pl.BlockSpec((2 * HID, 4 * HID), lambda t: (0, 0)),
        pl.BlockSpec((1, 4 * HID), lambda t: (0, 0)),
    ]
    args = (emb,) + tuple(weights)
    if not first:
        in_specs += [st_spec] * 4
        args += tuple(carry)
    return pl.pallas_call(
        _make_lstm_body(n_steps, first),
        grid=(n_steps,),
        in_specs=in_specs,
        out_specs=[st_spec] * 4,
        out_shape=[st] * 4,
        scratch_shapes=[pltpu.VMEM((B, HID), jnp.float32)] * 4,
        interpret=interpret,
    )(*args)


_TV = 4096


def _fc_body(htT_ref, w_ref, b_ref, out_ref):
    out_ref[...] = (
        jnp.dot(w_ref[...], htT_ref[...], preferred_element_type=jnp.float32)
        + jnp.transpose(b_ref[...])
    )


def _fc(htT, W_fc, b_fc2d, *, interpret=False):
    nv = pl.cdiv(N_VOCAB, _TV)
    return pl.pallas_call(
        _fc_body,
        grid=(nv,),
        in_specs=[
            pl.BlockSpec((HID, B), lambda v: (0, 0)),
            pl.BlockSpec((_TV, HID), lambda v: (v, 0)),
            pl.BlockSpec((1, _TV), lambda v: (0, v)),
        ],
        out_specs=pl.BlockSpec((_TV, B), lambda v: (v, 0)),
        out_shape=jax.ShapeDtypeStruct((N_VOCAB, B), jnp.float32),
        interpret=interpret,
    )(htT, W_fc, b_fc2d)


def kernel(seq_in, embeddings, W_ih0, W_hh0, b_ih0, b_hh0,
           W_ih1, W_hh1, b_ih1, b_hh1, W_fc, b_fc):
    idx_flat = seq_in.T.reshape(-1).astype(jnp.int32)
    table_p = jnp.pad(embeddings, ((0, 0), (0, _D - EMB)))
    emb1 = _sc_gather(idx_flat[: _LH * B], table_p).reshape(_LH, B, _D)
    emb2 = _sc_gather(idx_flat[_LH * B:], table_p).reshape(L - _LH, B, _D)

    b0 = (b_ih0 + b_hh0).reshape(1, 4 * HID)
    b1 = (b_ih1 + b_hh1).reshape(1, 4 * HID)
    bf = jnp.bfloat16
    w0cat = jnp.concatenate([W_ih0.T, W_hh0.T], axis=0).astype(bf)
    w1cat = jnp.concatenate([W_ih1.T, W_hh1.T], axis=0).astype(bf)
    weights = (w0cat, b0, w1cat, b1)
    carry = _lstm_part(emb1, weights)
    carry = _lstm_part(emb2, weights, carry)
    ht = carry[2]

    outT = _fc(ht.T, W_fc, b_fc.reshape(1, N_VOCAB))
    return outT.T


# speedup vs baseline: 2.4836x; 1.0069x over previous
import functools

import jax
import jax.numpy as jnp
from jax import lax
from jax.experimental import pallas as pl
from jax.experimental.pallas import tpu as pltpu
from jax.experimental.pallas import tpu_sc as plsc

N_VOCAB = 100000
HID = 128
EMB = 64
B = 1024
L = 50
NTOK = B * L


_D = 128


def _sc_gather(idx_flat, table_p):
    n_tok = idx_flat.shape[0]
    info = plsc.get_sparse_core_info()
    NC, NS = info.num_cores, info.num_subcores
    NW = NC * NS
    n_per_w = n_tok // NW
    CH = 80
    NCH = n_per_w // CH
    HALF = NCH // 2
    assert NCH * CH == n_per_w and n_tok % NW == 0 and NCH % 2 == 0

    mesh = plsc.VectorSubcoreMesh(core_axis_name="c", subcore_axis_name="s")

    @functools.partial(
        pl.kernel,
        out_type=jax.ShapeDtypeStruct((n_tok, _D), jnp.float32),
        mesh=mesh,
        scratch_types=[
            pltpu.VMEM((NCH, CH), jnp.int32),
            pltpu.VMEM((HALF * CH, _D), jnp.float32),
            pltpu.SemaphoreType.DMA,
        ],
        compiler_params=pltpu.CompilerParams(use_tc_tiling_on_sc=True),
    )
    def gather_k(idx_hbm, table_hbm, out_hbm, idx_v, rows_v, sem):
        wid = lax.axis_index("s") * NC + lax.axis_index("c")
        base = wid * n_per_w
        pltpu.sync_copy(idx_hbm.at[wid], idx_v)
        for h in range(2):
            copies = []
            for j in range(HALF):
                copies.append(
                    pltpu.async_copy(
                        table_hbm.at[idx_v.at[h * HALF + j]],
                        rows_v.at[pl.ds(j * CH, CH)],
                        sem,
                    )
                )
            for c in copies:
                c.wait()
            pltpu.sync_copy(
                rows_v, out_hbm.at[pl.ds(base + h * HALF * CH, HALF * CH)]
            )

    return gather_k(idx_flat.reshape(NW, NCH, CH), table_p)


def _sig(x):
    return 0.5 * jnp.tanh(0.5 * x) + 0.5


_LH = 15


def _make_lstm_body(n_steps, first):
    def body(*refs):
        if first:
            (x_ref, wih0_ref, b0_ref, wih1_ref, b1_ref,
             oh0, oc0, oh1, oc1, h0, c0, h1, c1) = refs
        else:
            (x_ref, wih0_ref, b0_ref, wih1_ref, b1_ref,
             ih0, ic0, ih1, ic1,
             oh0, oc0, oh1, oc1, h0, c0, h1, c1) = refs
        t = pl.program_id(0)

        @pl.when(t == 0)
        def _():
            if first:
                h0[...] = jnp.zeros_like(h0)
                c0[...] = jnp.zeros_like(c0)
                h1[...] = jnp.zeros_like(h1)
                c1[...] = jnp.zeros_like(c1)
            else:
                h0[...] = ih0[...]
                c0[...] = ic0[...]
                h1[...] = ih1[...]
                c1[...] = ic1[...]

        x = x_ref[0][:, :EMB].astype(jnp.bfloat16)
        xh = jnp.concatenate([x, h0[...].astype(jnp.bfloat16)], axis=1)
        g0 = (
            jnp.dot(xh, wih0_ref[...], preferred_element_type=jnp.float32)
            + b0_ref[...]
        )
        i0 = _sig(g0[:, 0:HID])
        f0 = _sig(g0[:, HID:2 * HID])
        gg0 = jnp.tanh(g0[:, 2 * HID:3 * HID])
        o0 = _sig(g0[:, 3 * HID:4 * HID])
        cn0 = f0 * c0[...] + i0 * gg0
        hn0 = o0 * jnp.tanh(cn0)
        c0[...] = cn0
        h0[...] = hn0

        hh = jnp.concatenate(
            [hn0.astype(jnp.bfloat16), h1[...].astype(jnp.bfloat16)], axis=1)
        g1 = (
            jnp.dot(hh, wih1_ref[...], preferred_element_type=jnp.float32)
            + b1_ref[...]
        )
        i1 = _sig(g1[:, 0:HID])
        f1 = _sig(g1[:, HID:2 * HID])
        gg1 = jnp.tanh(g1[:, 2 * HID:3 * HID])
        o1 = _sig(g1[:, 3 * HID:4 * HID])
        cn1 = f1 * c1[...] + i1 * gg1
        hn1 = o1 * jnp.tanh(cn1)
        c1[...] = cn1
        h1[...] = hn1

        @pl.when(t == n_steps - 1)
        def _():
            oh0[...] = hn0
            oc0[...] = cn0
            oh1[...] = hn1
            oc1[...] = cn1

    return body


def _lstm_part(emb, weights, carry=None, *, interpret=False):
    n_steps = emb.shape[0]
    first = carry is None
    st = jax.ShapeDtypeStruct((B, HID), jnp.float32)
    st_spec = pl.BlockSpec((B, HID), lambda t: (0, 0))
    in_specs = [
        pl.BlockSpec((1, B, _D), lambda t: (t, 0, 0)),
        pl.BlockSpec((EMB + HID, 4 * HID), lambda t: (0, 0)),
        pl.BlockSpec((1, 4 * HID), lambda t: (0, 0)),
        pl.BlockSpec((2 * HID, 4 * HID), lambda t: (0, 0)),
        pl.BlockSpec((1, 4 * HID), lambda t: (0, 0)),
    ]
    args = (emb,) + tuple(weights)
    if not first:
        in_specs += [st_spec] * 4
        args += tuple(carry)
    return pl.pallas_call(
        _make_lstm_body(n_steps, first),
        grid=(n_steps,),
        in_specs=in_specs,
        out_specs=[st_spec] * 4,
        out_shape=[st] * 4,
        scratch_shapes=[pltpu.VMEM((B, HID), jnp.float32)] * 4,
        interpret=interpret,
    )(*args)


_TV = 4096


def _fc_body(htT_ref, w_ref, b_ref, out_ref):
    out_ref[...] = (
        jnp.dot(w_ref[...], htT_ref[...], preferred_element_type=jnp.float32)
        + jnp.transpose(b_ref[...])
    )


def _fc(htT, W_fc, b_fc2d, *, interpret=False):
    nv = pl.cdiv(N_VOCAB, _TV)
    return pl.pallas_call(
        _fc_body,
        grid=(nv,),
        in_specs=[
            pl.BlockSpec((HID, B), lambda v: (0, 0)),
            pl.BlockSpec((_TV, HID), lambda v: (v, 0)),
            pl.BlockSpec((1, _TV), lambda v: (0, v)),
        ],
        out_specs=pl.BlockSpec((_TV, B), lambda v: (v, 0)),
        out_shape=jax.ShapeDtypeStruct((N_VOCAB, B), jnp.float32),
        interpret=interpret,
    )(htT, W_fc, b_fc2d)


def kernel(seq_in, embeddings, W_ih0, W_hh0, b_ih0, b_hh0,
           W_ih1, W_hh1, b_ih1, b_hh1, W_fc, b_fc):
    idx_flat = seq_in.T.reshape(-1).astype(jnp.int32)
    table_p = jnp.pad(embeddings, ((0, 0), (0, _D - EMB)))
    emb1 = _sc_gather(idx_flat[: _LH * B], table_p).reshape(_LH, B, _D)
    emb2 = _sc_gather(idx_flat[_LH * B:], table_p).reshape(L - _LH, B, _D)

    b0 = (b_ih0 + b_hh0).reshape(1, 4 * HID)
    b1 = (b_ih1 + b_hh1).reshape(1, 4 * HID)
    bf = jnp.bfloat16
    w0cat = jnp.concatenate([W_ih0.T, W_hh0.T], axis=0).astype(bf)
    w1cat = jnp.concatenate([W_ih1.T, W_hh1.T], axis=0).astype(bf)
    weights = (w0cat, b0, w1cat, b1)
    carry = _lstm_part(emb1, weights)
    carry = _lstm_part(emb2, weights, carry)
    ht = carry[2]

    outT = _fc(ht.T, W_fc, b_fc.reshape(1, N_VOCAB))
    return outT.T
